# Initial kernel scaffold; baseline (speedup 1.0000x reference)
import jax
import jax.numpy as jnp
from jax.experimental import pallas as pl


def kernel(x, duty):
    raise NotImplementedError("write your pallas kernel here")


# speedup vs baseline: 9.5041x; 9.5041x over previous
import functools

import jax
import jax.numpy as jnp
from jax.experimental import pallas as pl

_K = 512
_ALPHA = 0.01
_GAMMA = 1.0


def _sortable(x):
    s = jax.lax.bitcast_convert_type(x, jnp.int32)
    return s ^ ((s >> 31) & jnp.int32(0x7FFFFFFF))


def _thr_kernel(x_ref, thr_ref, cc_ref):
    key = _sortable(x_ref[...])
    r = key.shape[0]

    def body(_, carry):
        lo, hi = carry
        mid = (lo >> 1) + (hi >> 1) + (lo & hi & 1)
        cnt = jnp.sum((key >= mid).astype(jnp.int32), axis=1, keepdims=True)
        ge = cnt >= _K
        return jnp.where(ge, mid, lo), jnp.where(ge, hi, mid)

    lo0 = jnp.full((r, 1), jnp.iinfo(jnp.int32).min, jnp.int32)
    hi0 = jnp.full((r, 1), jnp.iinfo(jnp.int32).max, jnp.int32)
    lo, _ = jax.lax.fori_loop(0, 32, body, (lo0, hi0))
    thr_ref[...] = lo
    mask = key >= lo
    cc_ref[...] = jnp.sum(mask.astype(jnp.float32), axis=0, keepdims=True)[None]


def _out_kernel(target, x_ref, thr_ref, cc_ref, duty_ref, out_ref):
    x = x_ref[...]
    cc = jnp.sum(cc_ref[...][:, 0, :], axis=0, keepdims=True)
    duty_new = duty_ref[...] * (1.0 - _ALPHA) + (_ALPHA / x.shape[0]) * cc
    boost = jnp.exp(-_GAMMA * (duty_new - target))
    mask = _sortable(x) >= thr_ref[...]
    out_ref[...] = jnp.where(mask, x * boost, 0.0)


def kernel(x, duty):
    b, d = x.shape
    rb = 8
    nrb = b // rb
    cb = 2048
    ncb = d // cb
    target = _K / d

    thr, cc = pl.pallas_call(
        _thr_kernel,
        grid=(nrb,),
        in_specs=[pl.BlockSpec((rb, d), lambda i: (i, 0))],
        out_specs=[
            pl.BlockSpec((rb, 1), lambda i: (i, 0)),
            pl.BlockSpec((1, 1, d), lambda i: (i, 0, 0)),
        ],
        out_shape=[
            jax.ShapeDtypeStruct((b, 1), jnp.int32),
            jax.ShapeDtypeStruct((nrb, 1, d), jnp.float32),
        ],
    )(x)

    out = pl.pallas_call(
        functools.partial(_out_kernel, target),
        grid=(ncb,),
        in_specs=[
            pl.BlockSpec((b, cb), lambda j: (0, j)),
            pl.BlockSpec((b, 1), lambda j: (0, 0)),
            pl.BlockSpec((nrb, 1, cb), lambda j: (0, 0, j)),
            pl.BlockSpec((1, cb), lambda j: (0, j)),
        ],
        out_specs=pl.BlockSpec((b, cb), lambda j: (0, j)),
        out_shape=jax.ShapeDtypeStruct((b, d), jnp.float32),
    )(x, thr, cc, duty)
    return out
